# baseline (speedup 1.0000x reference)
import functools

import jax
import jax.numpy as jnp
from jax import lax
from jax.experimental import pallas as pl
from jax.experimental.pallas import tpu as pltpu
from jax.experimental.pallas import tpu_sc as plsc

NC = 2
NS = 16
NW = NC * NS
CHUNK = 128
DEGW = 16


def _sc_mesh():
    return plsc.VectorSubcoreMesh(
        core_axis_name="c", subcore_axis_name="s", num_cores=NC, num_subcores=NS
    )


def _row_split(n_rows):
    per = ((n_rows // NS + 7) // 8) * 8
    last = n_rows - (NS - 1) * per
    return per, last


def _sc_degree(dst, n_nodes):
    E = dst.shape[0]
    EW = E // NW
    DCH = 2000
    n_ch = EW // DCH
    PR = -(-n_nodes // 128)
    PRP = ((PR + 15) // 16) * 16
    rows_per_tile = PRP // NS

    @functools.partial(
        pl.kernel,
        out_type=jax.ShapeDtypeStruct((NC * PRP, 128), jnp.float32),
        mesh=_sc_mesh(),
        scratch_types=[
            pltpu.VMEM((DCH,), jnp.int32),
            pltpu.VMEM((PRP, 128), jnp.float32),
            pltpu.VMEM((PRP,), jnp.int32),
            pltpu.VMEM_SHARED((PRP, 128), jnp.float32),
        ],
        compiler_params=pltpu.CompilerParams(needs_layout_passes=False),
    )
    def k(dst_h, out_h, didx, hist, rowidx, acc):
        c = lax.axis_index("c")
        s = lax.axis_index("s")
        base = (c * NS + s) * EW

        zero16 = jnp.zeros((16,), jnp.float32)

        def zero_body(i, carry):
            hist[i >> 3, pl.ds((i & 7) * 16, 16)] = zero16
            return carry

        lax.fori_loop(0, PRP * 8, zero_body, 0)

        def iota_body(i, carry):
            rowidx[pl.ds(i * 16, 16)] = lax.iota(jnp.int32, 16) + i * 16
            return carry

        lax.fori_loop(0, PRP // 16, iota_body, 0)

        @pl.when(s == 0)
        def _():
            pltpu.sync_copy(hist, acc)

        plsc.subcore_barrier()

        def chunk_body(i, carry):
            off = pl.multiple_of(base + i * DCH, 8)
            pltpu.sync_copy(dst_h.at[pl.ds(off, DCH)], didx)

            def vec_body(j, carry2):
                idx = didx[pl.ds(j * 16, 16)]
                cnt, last = plsc.scan_count(idx)
                plsc.addupdate_scatter(
                    hist,
                    [lax.shift_right_logical(idx, 7), lax.bitwise_and(idx, 127)],
                    cnt.astype(jnp.float32),
                    mask=last,
                )
                return carry2

            lax.fori_loop(0, DCH // 16, vec_body, 0)
            return carry

        lax.fori_loop(0, n_ch, chunk_body, 0)

        pltpu.sync_copy(hist, acc.at[rowidx], add=True)
        plsc.subcore_barrier()

        ndump = PRP // 8

        @pl.when(s < ndump)
        def _():
            r0 = s * 8
            pltpu.sync_copy(
                acc.at[pl.ds(r0, 8)], out_h.at[pl.ds(c * PRP + r0, 8)]
            )

    return k(dst), PRP


def _sc_aggregate(src, dst, y, zeros, n_nodes):
    E = src.shape[0]
    D = y.shape[1]
    EW = E // NW
    n_full = EW // CHUNK
    tail = EW % CHUNK
    per, last = _row_split(n_nodes)

    SEG0 = 40 * CHUNK if EW >= 40 * CHUNK else (EW // (2 * CHUNK)) * CHUNK
    rest = ((EW - SEG0) // CHUNK) * CHUNK
    segs = [n for n in (SEG0, rest) if n > 0]
    tail = EW - sum(segs)

    @functools.partial(
        pl.kernel,
        out_type=jax.ShapeDtypeStruct((NC * n_nodes, D), jnp.float32),
        mesh=_sc_mesh(),
        scratch_types=[
            pltpu.VMEM((max(segs),), jnp.int32),
            pltpu.VMEM((max(segs),), jnp.int32),
            pltpu.VMEM((max(tail, 8),), jnp.int32),
            pltpu.VMEM((max(tail, 8),), jnp.int32),
            pltpu.VMEM((CHUNK, D), jnp.float32),
            pltpu.VMEM((CHUNK, D), jnp.float32),
            pltpu.VMEM((max(tail, 8), D), jnp.float32),
            pltpu.VMEM_SHARED((n_nodes, D), jnp.float32),
            pltpu.SemaphoreType.DMA,
            pltpu.SemaphoreType.DMA,
            pltpu.SemaphoreType.DMA,
            pltpu.SemaphoreType.DMA,
        ],
    )
    def k(src_h, dst_h, y_h, zeros_h, out_h,
          sall, dall, sidx_t, didx_t, rows0, rows1, rows_t,
          acc, sg0, sg1, ss0, ss1):
        c = lax.axis_index("c")
        s = lax.axis_index("s")
        base = (c * NS + s) * EW
        r0 = s * per

        @pl.when(s < NS - 1)
        def _():
            pltpu.sync_copy(zeros_h.at[pl.ds(r0, per)], acc.at[pl.ds(r0, per)])

        @pl.when(s == NS - 1)
        def _():
            pltpu.sync_copy(
                zeros_h.at[pl.ds((NS - 1) * per, last)],
                acc.at[pl.ds((NS - 1) * per, last)],
            )

        plsc.subcore_barrier()

        def sidx(i):
            return sall.at[pl.ds(pl.multiple_of(i * CHUNK, 8), CHUNK)]

        def didx(i):
            return dall.at[pl.ds(pl.multiple_of(i * CHUNK, 8), CHUNK)]

        seg_off = 0
        for seg_n in segs:
            boff = pl.multiple_of(base + seg_off, 8)
            pltpu.sync_copy(src_h.at[pl.ds(boff, seg_n)], sall.at[pl.ds(0, seg_n)])
            pltpu.sync_copy(dst_h.at[pl.ds(boff, seg_n)], dall.at[pl.ds(0, seg_n)])
            n_pair = seg_n // (2 * CHUNK)

            pltpu.async_copy(y_h.at[sidx(0)], rows0, sg0)

            def body(j, carry):
                a = 2 * j

                @pl.when(j > 0)
                def _():
                    pltpu.make_async_copy(rows1, acc.at[didx(a - 1)], ss1).wait()

                pltpu.make_async_copy(y_h.at[sidx(a)], rows0, sg0).wait()
                pltpu.async_copy(y_h.at[sidx(a + 1)], rows1, sg1)
                pltpu.async_copy(rows0, acc.at[didx(a)], ss0, add=True)
                pltpu.make_async_copy(y_h.at[sidx(a + 1)], rows1, sg1).wait()
                pltpu.async_copy(rows1, acc.at[didx(a + 1)], ss1, add=True)
                pltpu.make_async_copy(rows0, acc.at[didx(a)], ss0).wait()

                @pl.when(j < n_pair - 1)
                def _():
                    pltpu.async_copy(y_h.at[sidx(a + 2)], rows0, sg0)

                return carry

            lax.fori_loop(0, n_pair, body, 0)
            pltpu.make_async_copy(
                rows1, acc.at[didx(seg_n // CHUNK - 1)], ss1
            ).wait()
            seg_off += seg_n

        if tail:
            off = pl.multiple_of(base + seg_off, 8)
            pltpu.sync_copy(src_h.at[pl.ds(off, tail)], sidx_t.at[pl.ds(0, tail)])
            pltpu.sync_copy(dst_h.at[pl.ds(off, tail)], didx_t.at[pl.ds(0, tail)])
            pltpu.async_copy(
                y_h.at[sidx_t.at[pl.ds(0, tail)]], rows_t.at[pl.ds(0, tail)], sg0
            ).wait()
            pltpu.sync_copy(
                rows_t.at[pl.ds(0, tail)], acc.at[didx_t.at[pl.ds(0, tail)]], add=True
            )

        plsc.subcore_barrier()

        @pl.when(s < NS - 1)
        def _():
            pltpu.sync_copy(
                acc.at[pl.ds(r0, per)], out_h.at[pl.ds(c * n_nodes + r0, per)]
            )

        @pl.when(s == NS - 1)
        def _():
            pltpu.sync_copy(
                acc.at[pl.ds((NS - 1) * per, last)],
                out_h.at[pl.ds(c * n_nodes + (NS - 1) * per, last)],
            )

    return k(src, dst, y, zeros)


def _tc_prescale(deg16, x):
    n, d = x.shape
    R = 1000

    def body(deg_ref, x_ref, y_ref):
        dinv = lax.rsqrt(deg_ref[...] + 1.0)
        y_ref[...] = x_ref[...] * dinv[:, 0:1]

    return pl.pallas_call(
        body,
        grid=(n // R,),
        in_specs=[
            pl.BlockSpec((R, DEGW), lambda i: (i, 0)),
            pl.BlockSpec((R, d), lambda i: (i, 0)),
        ],
        out_specs=pl.BlockSpec((R, d), lambda i: (i, 0)),
        out_shape=jax.ShapeDtypeStruct((n, d), jnp.float32),
    )(deg16, x)


def _tc_dense(s_p, y, deg16, h, Mz, Bz, Mr, Br, Mh, Bh, Wout, bv):
    n, d = y.shape
    R = 1000

    def body(s_ref, y_ref, deg_ref, h_ref,
             Mz_ref, Bz_ref, Mr_ref, Br_ref, Mh_ref, Bh_ref, Wout_ref, bv_ref,
             z_ref, h0_ref):
        dinv = lax.rsqrt(deg_ref[...] + 1.0)[:, 0:1]
        hh = h_ref[...]
        agg = (s_ref[0] + s_ref[1] + y_ref[...]) * dinv
        dot = lambda a, b: jnp.dot(a, b, preferred_element_type=jnp.float32)
        Zg = jax.nn.sigmoid(dot(agg, Mz_ref[...]) + dot(hh, Bz_ref[...]) + bv_ref[0:1])
        Rg = jax.nn.sigmoid(dot(agg, Mr_ref[...]) + dot(hh, Br_ref[...]) + bv_ref[1:2])
        Ht = jnp.tanh(dot(agg, Mh_ref[...]) + dot(hh * Rg, Bh_ref[...]) + bv_ref[2:3])
        h0 = Zg * hh + (1.0 - Zg) * Ht
        h0_ref[...] = h0
        z_ref[...] = dot(jnp.maximum(h0, 0.0), Wout_ref[...]) + bv_ref[3:4]

    wspec = pl.BlockSpec((d, d), lambda i: (0, 0))
    return pl.pallas_call(
        body,
        grid=(n // R,),
        in_specs=[
            pl.BlockSpec((2, R, d), lambda i: (0, i, 0)),
            pl.BlockSpec((R, d), lambda i: (i, 0)),
            pl.BlockSpec((R, DEGW), lambda i: (i, 0)),
            pl.BlockSpec((R, d), lambda i: (i, 0)),
            wspec, wspec, wspec, wspec, wspec, wspec, wspec,
            pl.BlockSpec((4, d), lambda i: (0, 0)),
        ],
        out_specs=[
            pl.BlockSpec((R, d), lambda i: (i, 0)),
            pl.BlockSpec((R, d), lambda i: (i, 0)),
        ],
        out_shape=[
            jax.ShapeDtypeStruct((n, d), jnp.float32),
            jax.ShapeDtypeStruct((n, d), jnp.float32),
        ],
    )(s_p, y, deg16, h, Mz, Bz, Mr, Br, Mh, Bh, Wout, bv)


def kernel(node_feat, h, src, dst, Wz, bz, Wr, br, Wh, bh,
           Wlz, blz, Wlr, blr, Wlh, blh, Wout, bout):
    n, d = node_feat.shape

    zeros = jnp.zeros((n, d), jnp.float32)

    deg_pk, prp = _sc_degree(dst, n)
    deg = deg_pk.reshape(NC, prp * 128)[:, :n].sum(axis=0)
    deg16 = jnp.broadcast_to(deg[:, None], (n, DEGW))

    y = _tc_prescale(deg16, node_feat)

    s_flat = _sc_aggregate(src, dst, y, zeros, n)
    s_p = s_flat.reshape(NC, n, d)

    Mz = Wz @ Wlz[:d]
    Mr = Wr @ Wlr[:d]
    Mh = Wh @ Wlh[:d]
    bv = jnp.stack([
        bz @ Wlz[:d] + blz,
        br @ Wlr[:d] + blr,
        bh @ Wlh[:d] + blh,
        bout,
    ])

    z, h0 = _tc_dense(s_p, y, deg16, h, Mz, Wlz[d:], Mr, Wlr[d:], Mh, Wlh[d:],
                      Wout, bv)
    return (z, h0)


# speedup vs baseline: 1.0179x; 1.0179x over previous
import functools

import jax
import jax.numpy as jnp
from jax import lax
from jax.experimental import pallas as pl
from jax.experimental.pallas import tpu as pltpu
from jax.experimental.pallas import tpu_sc as plsc

NC = 2
NS = 16
NW = NC * NS
CHUNK = 128
DEGW = 16


def _sc_mesh():
    return plsc.VectorSubcoreMesh(
        core_axis_name="c", subcore_axis_name="s", num_cores=NC, num_subcores=NS
    )


def _row_split(n_rows):
    per = ((n_rows // NS + 7) // 8) * 8
    last = n_rows - (NS - 1) * per
    return per, last


def _sc_degree(dst, n_nodes):
    E = dst.shape[0]
    EW = E // NW
    UNROLL = 5
    n_vec = EW // 16
    n_grp = n_vec // UNROLL
    rem_vec = n_vec % UNROLL
    PR = -(-n_nodes // 128)
    PRP = ((PR + 15) // 16) * 16
    rows_per_tile = PRP // NS

    @functools.partial(
        pl.kernel,
        out_type=jax.ShapeDtypeStruct((NC * PRP, 128), jnp.float32),
        mesh=_sc_mesh(),
        scratch_types=[
            pltpu.VMEM((EW,), jnp.int32),
            pltpu.VMEM((PRP, 128), jnp.float32),
            pltpu.VMEM((PRP,), jnp.int32),
            pltpu.VMEM_SHARED((PRP, 128), jnp.float32),
            pltpu.SemaphoreType.DMA,
        ],
        compiler_params=pltpu.CompilerParams(needs_layout_passes=False),
    )
    def k(dst_h, out_h, didx, hist, rowidx, acc, sem):
        c = lax.axis_index("c")
        s = lax.axis_index("s")
        base = (c * NS + s) * EW

        pltpu.async_copy(dst_h.at[pl.ds(pl.multiple_of(base, 8), EW)], didx, sem)

        zero16 = jnp.zeros((16,), jnp.float32)

        def zero_body(i, carry):
            hist[i >> 3, pl.ds((i & 7) * 16, 16)] = zero16
            return carry

        lax.fori_loop(0, PRP * 8, zero_body, 0)

        def iota_body(i, carry):
            rowidx[pl.ds(i * 16, 16)] = lax.iota(jnp.int32, 16) + i * 16
            return carry

        lax.fori_loop(0, PRP // 16, iota_body, 0)

        @pl.when(s == 0)
        def _():
            pltpu.sync_copy(hist, acc)

        plsc.subcore_barrier()
        pltpu.make_async_copy(
            dst_h.at[pl.ds(pl.multiple_of(base, 8), EW)], didx, sem
        ).wait()

        def hist_add(j):
            idx = didx[pl.ds(pl.multiple_of(j * 16, 16), 16)]
            cnt, last = plsc.scan_count(idx)
            plsc.addupdate_scatter(
                hist,
                [lax.shift_right_logical(idx, 7), lax.bitwise_and(idx, 127)],
                cnt.astype(jnp.float32),
                mask=last,
            )

        def grp_body(g, carry):
            for u in range(UNROLL):
                hist_add(g * UNROLL + u)
            return carry

        lax.fori_loop(0, n_grp, grp_body, 0)
        for j in range(n_grp * UNROLL, n_vec):
            hist_add(j)

        pltpu.sync_copy(hist, acc.at[rowidx], add=True)
        plsc.subcore_barrier()

        ndump = PRP // 8

        @pl.when(s < ndump)
        def _():
            r0 = s * 8
            pltpu.sync_copy(
                acc.at[pl.ds(r0, 8)], out_h.at[pl.ds(c * PRP + r0, 8)]
            )

    return k(dst), PRP


def _sc_aggregate(src, dst, y, zeros, n_nodes):
    E = src.shape[0]
    D = y.shape[1]
    EW = E // NW
    n_full = EW // CHUNK
    tail = EW % CHUNK
    per, last = _row_split(n_nodes)

    SEG0 = 40 * CHUNK if EW >= 40 * CHUNK else (EW // (2 * CHUNK)) * CHUNK
    rest = ((EW - SEG0) // CHUNK) * CHUNK
    segs = [n for n in (SEG0, rest) if n > 0]
    tail = EW - sum(segs)

    @functools.partial(
        pl.kernel,
        out_type=jax.ShapeDtypeStruct((NC * n_nodes, D), jnp.float32),
        mesh=_sc_mesh(),
        scratch_types=[
            pltpu.VMEM((max(segs),), jnp.int32),
            pltpu.VMEM((max(segs),), jnp.int32),
            pltpu.VMEM((max(tail, 8),), jnp.int32),
            pltpu.VMEM((max(tail, 8),), jnp.int32),
            pltpu.VMEM((CHUNK, D), jnp.float32),
            pltpu.VMEM((CHUNK, D), jnp.float32),
            pltpu.VMEM((max(tail, 8), D), jnp.float32),
            pltpu.VMEM_SHARED((n_nodes, D), jnp.float32),
            pltpu.SemaphoreType.DMA,
            pltpu.SemaphoreType.DMA,
            pltpu.SemaphoreType.DMA,
            pltpu.SemaphoreType.DMA,
        ],
    )
    def k(src_h, dst_h, y_h, zeros_h, out_h,
          sall, dall, sidx_t, didx_t, rows0, rows1, rows_t,
          acc, sg0, sg1, ss0, ss1):
        c = lax.axis_index("c")
        s = lax.axis_index("s")
        base = (c * NS + s) * EW
        r0 = s * per

        @pl.when(s < NS - 1)
        def _():
            pltpu.sync_copy(zeros_h.at[pl.ds(r0, per)], acc.at[pl.ds(r0, per)])

        @pl.when(s == NS - 1)
        def _():
            pltpu.sync_copy(
                zeros_h.at[pl.ds((NS - 1) * per, last)],
                acc.at[pl.ds((NS - 1) * per, last)],
            )

        plsc.subcore_barrier()

        def sidx(i):
            return sall.at[pl.ds(pl.multiple_of(i * CHUNK, 8), CHUNK)]

        def didx(i):
            return dall.at[pl.ds(pl.multiple_of(i * CHUNK, 8), CHUNK)]

        seg_off = 0
        for seg_n in segs:
            boff = pl.multiple_of(base + seg_off, 8)
            pltpu.sync_copy(src_h.at[pl.ds(boff, seg_n)], sall.at[pl.ds(0, seg_n)])
            pltpu.sync_copy(dst_h.at[pl.ds(boff, seg_n)], dall.at[pl.ds(0, seg_n)])
            n_pair = seg_n // (2 * CHUNK)

            pltpu.async_copy(y_h.at[sidx(0)], rows0, sg0)

            def body(j, carry):
                a = 2 * j

                @pl.when(j > 0)
                def _():
                    pltpu.make_async_copy(rows1, acc.at[didx(a - 1)], ss1).wait()

                pltpu.make_async_copy(y_h.at[sidx(a)], rows0, sg0).wait()
                pltpu.async_copy(y_h.at[sidx(a + 1)], rows1, sg1)
                pltpu.async_copy(rows0, acc.at[didx(a)], ss0, add=True)
                pltpu.make_async_copy(y_h.at[sidx(a + 1)], rows1, sg1).wait()
                pltpu.async_copy(rows1, acc.at[didx(a + 1)], ss1, add=True)
                pltpu.make_async_copy(rows0, acc.at[didx(a)], ss0).wait()

                @pl.when(j < n_pair - 1)
                def _():
                    pltpu.async_copy(y_h.at[sidx(a + 2)], rows0, sg0)

                return carry

            lax.fori_loop(0, n_pair, body, 0)
            pltpu.make_async_copy(
                rows1, acc.at[didx(seg_n // CHUNK - 1)], ss1
            ).wait()
            seg_off += seg_n

        if tail:
            off = pl.multiple_of(base + seg_off, 8)
            pltpu.sync_copy(src_h.at[pl.ds(off, tail)], sidx_t.at[pl.ds(0, tail)])
            pltpu.sync_copy(dst_h.at[pl.ds(off, tail)], didx_t.at[pl.ds(0, tail)])
            pltpu.async_copy(
                y_h.at[sidx_t.at[pl.ds(0, tail)]], rows_t.at[pl.ds(0, tail)], sg0
            ).wait()
            pltpu.sync_copy(
                rows_t.at[pl.ds(0, tail)], acc.at[didx_t.at[pl.ds(0, tail)]], add=True
            )

        plsc.subcore_barrier()

        @pl.when(s < NS - 1)
        def _():
            pltpu.sync_copy(
                acc.at[pl.ds(r0, per)], out_h.at[pl.ds(c * n_nodes + r0, per)]
            )

        @pl.when(s == NS - 1)
        def _():
            pltpu.sync_copy(
                acc.at[pl.ds((NS - 1) * per, last)],
                out_h.at[pl.ds(c * n_nodes + (NS - 1) * per, last)],
            )

    return k(src, dst, y, zeros)


def _tc_prescale(deg16, x):
    n, d = x.shape
    R = 1000

    def body(deg_ref, x_ref, y_ref):
        dinv = lax.rsqrt(deg_ref[...] + 1.0)
        y_ref[...] = x_ref[...] * dinv[:, 0:1]

    return pl.pallas_call(
        body,
        grid=(n // R,),
        in_specs=[
            pl.BlockSpec((R, DEGW), lambda i: (i, 0)),
            pl.BlockSpec((R, d), lambda i: (i, 0)),
        ],
        out_specs=pl.BlockSpec((R, d), lambda i: (i, 0)),
        out_shape=jax.ShapeDtypeStruct((n, d), jnp.float32),
    )(deg16, x)


def _tc_dense(s_p, y, deg16, h, Mz, Bz, Mr, Br, Mh, Bh, Wout, bv):
    n, d = y.shape
    R = 1000

    def body(s_ref, y_ref, deg_ref, h_ref,
             Mz_ref, Bz_ref, Mr_ref, Br_ref, Mh_ref, Bh_ref, Wout_ref, bv_ref,
             z_ref, h0_ref):
        dinv = lax.rsqrt(deg_ref[...] + 1.0)[:, 0:1]
        hh = h_ref[...]
        agg = (s_ref[0] + s_ref[1] + y_ref[...]) * dinv
        dot = lambda a, b: jnp.dot(a, b, preferred_element_type=jnp.float32)
        Zg = jax.nn.sigmoid(dot(agg, Mz_ref[...]) + dot(hh, Bz_ref[...]) + bv_ref[0:1])
        Rg = jax.nn.sigmoid(dot(agg, Mr_ref[...]) + dot(hh, Br_ref[...]) + bv_ref[1:2])
        Ht = jnp.tanh(dot(agg, Mh_ref[...]) + dot(hh * Rg, Bh_ref[...]) + bv_ref[2:3])
        h0 = Zg * hh + (1.0 - Zg) * Ht
        h0_ref[...] = h0
        z_ref[...] = dot(jnp.maximum(h0, 0.0), Wout_ref[...]) + bv_ref[3:4]

    wspec = pl.BlockSpec((d, d), lambda i: (0, 0))
    return pl.pallas_call(
        body,
        grid=(n // R,),
        in_specs=[
            pl.BlockSpec((2, R, d), lambda i: (0, i, 0)),
            pl.BlockSpec((R, d), lambda i: (i, 0)),
            pl.BlockSpec((R, DEGW), lambda i: (i, 0)),
            pl.BlockSpec((R, d), lambda i: (i, 0)),
            wspec, wspec, wspec, wspec, wspec, wspec, wspec,
            pl.BlockSpec((4, d), lambda i: (0, 0)),
        ],
        out_specs=[
            pl.BlockSpec((R, d), lambda i: (i, 0)),
            pl.BlockSpec((R, d), lambda i: (i, 0)),
        ],
        out_shape=[
            jax.ShapeDtypeStruct((n, d), jnp.float32),
            jax.ShapeDtypeStruct((n, d), jnp.float32),
        ],
    )(s_p, y, deg16, h, Mz, Bz, Mr, Br, Mh, Bh, Wout, bv)


def kernel(node_feat, h, src, dst, Wz, bz, Wr, br, Wh, bh,
           Wlz, blz, Wlr, blr, Wlh, blh, Wout, bout):
    n, d = node_feat.shape

    zeros = jnp.zeros((n, d), jnp.float32)

    deg_pk, prp = _sc_degree(dst, n)
    deg = deg_pk.reshape(NC, prp * 128)[:, :n].sum(axis=0)
    deg16 = jnp.broadcast_to(deg[:, None], (n, DEGW))

    y = _tc_prescale(deg16, node_feat)

    s_flat = _sc_aggregate(src, dst, y, zeros, n)
    s_p = s_flat.reshape(NC, n, d)

    Mz = Wz @ Wlz[:d]
    Mr = Wr @ Wlr[:d]
    Mh = Wh @ Wlh[:d]
    bv = jnp.stack([
        bz @ Wlz[:d] + blz,
        br @ Wlr[:d] + blr,
        bh @ Wlh[:d] + blh,
        bout,
    ])

    z, h0 = _tc_dense(s_p, y, deg16, h, Mz, Wlz[d:], Mr, Wlr[d:], Mh, Wlh[d:],
                      Wout, bv)
    return (z, h0)


# speedup vs baseline: 1.0452x; 1.0269x over previous
import functools

import jax
import jax.numpy as jnp
from jax import lax
from jax.experimental import pallas as pl
from jax.experimental.pallas import tpu as pltpu
from jax.experimental.pallas import tpu_sc as plsc

NC = 2
NS = 16
NW = NC * NS
CHUNK = 128
DEGW = 16


def _sc_mesh():
    return plsc.VectorSubcoreMesh(
        core_axis_name="c", subcore_axis_name="s", num_cores=NC, num_subcores=NS
    )


def _row_split(n_rows):
    per = ((n_rows // NS + 7) // 8) * 8
    last = n_rows - (NS - 1) * per
    return per, last


def _sc_degree(dst, n_nodes):
    E = dst.shape[0]
    EW = E // NW
    UNROLL = 5
    n_vec = EW // 16
    n_grp = n_vec // UNROLL
    rem_vec = n_vec % UNROLL
    PR = -(-n_nodes // 128)
    PRP = ((PR + 15) // 16) * 16
    rows_per_tile = PRP // NS

    @functools.partial(
        pl.kernel,
        out_type=jax.ShapeDtypeStruct((NC * PRP, 128), jnp.float32),
        mesh=_sc_mesh(),
        scratch_types=[
            pltpu.VMEM((EW,), jnp.int32),
            pltpu.VMEM((PRP, 128), jnp.float32),
            pltpu.VMEM((PRP,), jnp.int32),
            pltpu.VMEM_SHARED((PRP, 128), jnp.float32),
            pltpu.SemaphoreType.DMA,
        ],
        compiler_params=pltpu.CompilerParams(needs_layout_passes=False),
    )
    def k(dst_h, out_h, didx, hist, rowidx, acc, sem):
        c = lax.axis_index("c")
        s = lax.axis_index("s")
        base = (c * NS + s) * EW

        pltpu.async_copy(dst_h.at[pl.ds(pl.multiple_of(base, 8), EW)], didx, sem)

        zero16 = jnp.zeros((16,), jnp.float32)

        def zero_body(i, carry):
            hist[i >> 3, pl.ds((i & 7) * 16, 16)] = zero16
            return carry

        lax.fori_loop(0, PRP * 8, zero_body, 0)

        def iota_body(i, carry):
            rowidx[pl.ds(i * 16, 16)] = lax.iota(jnp.int32, 16) + i * 16
            return carry

        lax.fori_loop(0, PRP // 16, iota_body, 0)

        @pl.when(s == 0)
        def _():
            pltpu.sync_copy(hist, acc)

        plsc.subcore_barrier()
        pltpu.make_async_copy(
            dst_h.at[pl.ds(pl.multiple_of(base, 8), EW)], didx, sem
        ).wait()

        def hist_add(j):
            idx = didx[pl.ds(pl.multiple_of(j * 16, 16), 16)]
            cnt, last = plsc.scan_count(idx)
            plsc.addupdate_scatter(
                hist,
                [lax.shift_right_logical(idx, 7), lax.bitwise_and(idx, 127)],
                cnt.astype(jnp.float32),
                mask=last,
            )

        def grp_body(g, carry):
            for u in range(UNROLL):
                hist_add(g * UNROLL + u)
            return carry

        lax.fori_loop(0, n_grp, grp_body, 0)
        for j in range(n_grp * UNROLL, n_vec):
            hist_add(j)

        pltpu.sync_copy(hist, acc.at[rowidx], add=True)
        plsc.subcore_barrier()

        ndump = PRP // 8

        @pl.when(s < ndump)
        def _():
            r0 = s * 8
            pltpu.sync_copy(
                acc.at[pl.ds(r0, 8)], out_h.at[pl.ds(c * PRP + r0, 8)]
            )

    return k(dst), PRP


def _sc_aggregate(src, dst, y, zeros, n_nodes):
    E = src.shape[0]
    D = y.shape[1]
    EW = E // NW
    n_full = EW // CHUNK
    tail = EW % CHUNK
    per, last = _row_split(n_nodes)

    SEG0 = 40 * CHUNK if EW >= 40 * CHUNK else (EW // (2 * CHUNK)) * CHUNK
    rest = ((EW - SEG0) // CHUNK) * CHUNK
    segs = [n for n in (SEG0, rest) if n > 0]
    tail = EW - sum(segs)

    @functools.partial(
        pl.kernel,
        out_type=jax.ShapeDtypeStruct((NC * n_nodes, D), jnp.float32),
        mesh=_sc_mesh(),
        scratch_types=[
            pltpu.VMEM((max(segs),), jnp.int32),
            pltpu.VMEM((max(segs),), jnp.int32),
            pltpu.VMEM((max(tail, 8),), jnp.int32),
            pltpu.VMEM((max(tail, 8),), jnp.int32),
            pltpu.VMEM((CHUNK, D), jnp.float32),
            pltpu.VMEM((CHUNK, D), jnp.float32),
            pltpu.VMEM((max(tail, 8), D), jnp.float32),
            pltpu.VMEM_SHARED((n_nodes, D), jnp.float32),
            pltpu.SemaphoreType.DMA,
            pltpu.SemaphoreType.DMA,
            pltpu.SemaphoreType.DMA,
            pltpu.SemaphoreType.DMA,
        ],
    )
    def k(src_h, dst_h, y_h, zeros_h, out_h,
          sall, dall, sidx_t, didx_t, rows0, rows1, rows_t,
          acc, sg0, sg1, ss0, ss1):
        c = lax.axis_index("c")
        s = lax.axis_index("s")
        base = (c * NS + s) * EW
        r0 = s * per

        @pl.when(s < NS - 1)
        def _():
            pltpu.sync_copy(zeros_h.at[pl.ds(r0, per)], acc.at[pl.ds(r0, per)])

        @pl.when(s == NS - 1)
        def _():
            pltpu.sync_copy(
                zeros_h.at[pl.ds((NS - 1) * per, last)],
                acc.at[pl.ds((NS - 1) * per, last)],
            )

        plsc.subcore_barrier()

        def sidx(i):
            return sall.at[pl.ds(pl.multiple_of(i * CHUNK, 8), CHUNK)]

        def didx(i):
            return dall.at[pl.ds(pl.multiple_of(i * CHUNK, 8), CHUNK)]

        seg_off = 0
        for seg_n in segs:
            boff = pl.multiple_of(base + seg_off, 8)
            pltpu.sync_copy(src_h.at[pl.ds(boff, seg_n)], sall.at[pl.ds(0, seg_n)])
            pltpu.sync_copy(dst_h.at[pl.ds(boff, seg_n)], dall.at[pl.ds(0, seg_n)])
            n_pair = seg_n // (2 * CHUNK)

            pltpu.async_copy(y_h.at[sidx(0)], rows0, sg0)

            def body(j, carry):
                a = 2 * j

                @pl.when(j > 0)
                def _():
                    pltpu.make_async_copy(rows1, acc.at[didx(a - 1)], ss1).wait()

                pltpu.make_async_copy(y_h.at[sidx(a)], rows0, sg0).wait()
                pltpu.async_copy(y_h.at[sidx(a + 1)], rows1, sg1)
                pltpu.async_copy(rows0, acc.at[didx(a)], ss0, add=True)
                pltpu.make_async_copy(y_h.at[sidx(a + 1)], rows1, sg1).wait()
                pltpu.async_copy(rows1, acc.at[didx(a + 1)], ss1, add=True)
                pltpu.make_async_copy(rows0, acc.at[didx(a)], ss0).wait()

                @pl.when(j < n_pair - 1)
                def _():
                    pltpu.async_copy(y_h.at[sidx(a + 2)], rows0, sg0)

                return carry

            lax.fori_loop(0, n_pair, body, 0)
            pltpu.make_async_copy(
                rows1, acc.at[didx(seg_n // CHUNK - 1)], ss1
            ).wait()
            seg_off += seg_n

        if tail:
            off = pl.multiple_of(base + seg_off, 8)
            pltpu.sync_copy(src_h.at[pl.ds(off, tail)], sidx_t.at[pl.ds(0, tail)])
            pltpu.sync_copy(dst_h.at[pl.ds(off, tail)], didx_t.at[pl.ds(0, tail)])
            pltpu.async_copy(
                y_h.at[sidx_t.at[pl.ds(0, tail)]], rows_t.at[pl.ds(0, tail)], sg0
            ).wait()
            pltpu.sync_copy(
                rows_t.at[pl.ds(0, tail)], acc.at[didx_t.at[pl.ds(0, tail)]], add=True
            )

        plsc.subcore_barrier()

        @pl.when(s < NS - 1)
        def _():
            pltpu.sync_copy(
                acc.at[pl.ds(r0, per)], out_h.at[pl.ds(c * n_nodes + r0, per)]
            )

        @pl.when(s == NS - 1)
        def _():
            pltpu.sync_copy(
                acc.at[pl.ds((NS - 1) * per, last)],
                out_h.at[pl.ds(c * n_nodes + (NS - 1) * per, last)],
            )

    return k(src, dst, y, zeros)


def _tc_prescale(deg16, x):
    n, d = x.shape
    R = 2000

    def body(deg_ref, x_ref, y_ref):
        dinv = lax.rsqrt(deg_ref[...] + 1.0)
        y_ref[...] = x_ref[...] * dinv[:, 0:1]

    return pl.pallas_call(
        body,
        grid=(n // R,),
        in_specs=[
            pl.BlockSpec((R, DEGW), lambda i: (i, 0)),
            pl.BlockSpec((R, d), lambda i: (i, 0)),
        ],
        out_specs=pl.BlockSpec((R, d), lambda i: (i, 0)),
        out_shape=jax.ShapeDtypeStruct((n, d), jnp.float32),
    )(deg16, x)


def _tc_dense(s_p, y, deg16, h, Mz, Bz, Mr, Br, Mh, Bh, Wout, bv):
    n, d = y.shape
    R = 2000

    def body(s_ref, y_ref, deg_ref, h_ref,
             Mz_ref, Bz_ref, Mr_ref, Br_ref, Mh_ref, Bh_ref, Wout_ref, bv_ref,
             z_ref, h0_ref):
        dinv = lax.rsqrt(deg_ref[...] + 1.0)[:, 0:1]
        hh = h_ref[...]
        agg = (s_ref[0] + s_ref[1] + y_ref[...]) * dinv
        dot = lambda a, b: jnp.dot(a, b, preferred_element_type=jnp.float32)
        Zg = jax.nn.sigmoid(dot(agg, Mz_ref[...]) + dot(hh, Bz_ref[...]) + bv_ref[0:1])
        Rg = jax.nn.sigmoid(dot(agg, Mr_ref[...]) + dot(hh, Br_ref[...]) + bv_ref[1:2])
        Ht = jnp.tanh(dot(agg, Mh_ref[...]) + dot(hh * Rg, Bh_ref[...]) + bv_ref[2:3])
        h0 = Zg * hh + (1.0 - Zg) * Ht
        h0_ref[...] = h0
        z_ref[...] = dot(jnp.maximum(h0, 0.0), Wout_ref[...]) + bv_ref[3:4]

    wspec = pl.BlockSpec((d, d), lambda i: (0, 0))
    return pl.pallas_call(
        body,
        grid=(n // R,),
        in_specs=[
            pl.BlockSpec((2, R, d), lambda i: (0, i, 0)),
            pl.BlockSpec((R, d), lambda i: (i, 0)),
            pl.BlockSpec((R, DEGW), lambda i: (i, 0)),
            pl.BlockSpec((R, d), lambda i: (i, 0)),
            wspec, wspec, wspec, wspec, wspec, wspec, wspec,
            pl.BlockSpec((4, d), lambda i: (0, 0)),
        ],
        out_specs=[
            pl.BlockSpec((R, d), lambda i: (i, 0)),
            pl.BlockSpec((R, d), lambda i: (i, 0)),
        ],
        out_shape=[
            jax.ShapeDtypeStruct((n, d), jnp.float32),
            jax.ShapeDtypeStruct((n, d), jnp.float32),
        ],
    )(s_p, y, deg16, h, Mz, Bz, Mr, Br, Mh, Bh, Wout, bv)


def kernel(node_feat, h, src, dst, Wz, bz, Wr, br, Wh, bh,
           Wlz, blz, Wlr, blr, Wlh, blh, Wout, bout):
    n, d = node_feat.shape

    zeros = jnp.zeros((n, d), jnp.float32)

    deg_pk, prp = _sc_degree(dst, n)
    deg = deg_pk.reshape(NC, prp * 128)[:, :n].sum(axis=0)
    deg16 = jnp.broadcast_to(deg[:, None], (n, DEGW))

    y = _tc_prescale(deg16, node_feat)

    s_flat = _sc_aggregate(src, dst, y, zeros, n)
    s_p = s_flat.reshape(NC, n, d)

    Mz = Wz @ Wlz[:d]
    Mr = Wr @ Wlr[:d]
    Mh = Wh @ Wlh[:d]
    bv = jnp.stack([
        bz @ Wlz[:d] + blz,
        br @ Wlr[:d] + blr,
        bh @ Wlh[:d] + blh,
        bout,
    ])

    z, h0 = _tc_dense(s_p, y, deg16, h, Mz, Wlz[d:], Mr, Wlr[d:], Mh, Wlh[d:],
                      Wout, bv)
    return (z, h0)


# speedup vs baseline: 1.0508x; 1.0053x over previous
import functools

import jax
import jax.numpy as jnp
from jax import lax
from jax.experimental import pallas as pl
from jax.experimental.pallas import tpu as pltpu
from jax.experimental.pallas import tpu_sc as plsc

NC = 2
NS = 16
NW = NC * NS
CHUNK = 128
DEGW = 16


def _sc_mesh():
    return plsc.VectorSubcoreMesh(
        core_axis_name="c", subcore_axis_name="s", num_cores=NC, num_subcores=NS
    )


def _row_split(n_rows):
    per = ((n_rows // NS + 7) // 8) * 8
    last = n_rows - (NS - 1) * per
    return per, last


def _sc_degree(dst, n_nodes):
    E = dst.shape[0]
    EW = E // NW
    UNROLL = 5
    n_vec = EW // 16
    n_grp = n_vec // UNROLL
    rem_vec = n_vec % UNROLL
    PR = -(-n_nodes // 128)
    PRP = ((PR + 15) // 16) * 16
    rows_per_tile = PRP // NS

    @functools.partial(
        pl.kernel,
        out_type=jax.ShapeDtypeStruct((NC * PRP, 128), jnp.float32),
        mesh=_sc_mesh(),
        scratch_types=[
            pltpu.VMEM((EW,), jnp.int32),
            pltpu.VMEM((PRP, 128), jnp.float32),
            pltpu.VMEM((PRP,), jnp.int32),
            pltpu.VMEM_SHARED((PRP, 128), jnp.float32),
            pltpu.SemaphoreType.DMA,
        ],
        compiler_params=pltpu.CompilerParams(needs_layout_passes=False),
    )
    def k(dst_h, out_h, didx, hist, rowidx, acc, sem):
        c = lax.axis_index("c")
        s = lax.axis_index("s")
        base = (c * NS + s) * EW

        pltpu.async_copy(dst_h.at[pl.ds(pl.multiple_of(base, 8), EW)], didx, sem)

        zero16 = jnp.zeros((16,), jnp.float32)

        def zero_body(i, carry):
            hist[i >> 3, pl.ds((i & 7) * 16, 16)] = zero16
            return carry

        lax.fori_loop(0, PRP * 8, zero_body, 0)

        def iota_body(i, carry):
            rowidx[pl.ds(i * 16, 16)] = lax.iota(jnp.int32, 16) + i * 16
            return carry

        lax.fori_loop(0, PRP // 16, iota_body, 0)

        @pl.when(s == 0)
        def _():
            pltpu.sync_copy(hist, acc)

        plsc.subcore_barrier()
        pltpu.make_async_copy(
            dst_h.at[pl.ds(pl.multiple_of(base, 8), EW)], didx, sem
        ).wait()

        def hist_add(j):
            idx = didx[pl.ds(pl.multiple_of(j * 16, 16), 16)]
            cnt, last = plsc.scan_count(idx)
            plsc.addupdate_scatter(
                hist,
                [lax.shift_right_logical(idx, 7), lax.bitwise_and(idx, 127)],
                cnt.astype(jnp.float32),
                mask=last,
            )

        def grp_body(g, carry):
            for u in range(UNROLL):
                hist_add(g * UNROLL + u)
            return carry

        lax.fori_loop(0, n_grp, grp_body, 0)
        for j in range(n_grp * UNROLL, n_vec):
            hist_add(j)

        pltpu.sync_copy(hist, acc.at[rowidx], add=True)
        plsc.subcore_barrier()

        ndump = PRP // 8

        @pl.when(s < ndump)
        def _():
            r0 = s * 8
            pltpu.sync_copy(
                acc.at[pl.ds(r0, 8)], out_h.at[pl.ds(c * PRP + r0, 8)]
            )

    return k(dst), PRP


def _sc_aggregate(src, dst, y, zeros, n_nodes):
    E = src.shape[0]
    D = y.shape[1]
    EW = E // NW
    n_full = EW // CHUNK
    tail = EW % CHUNK
    per, last = _row_split(n_nodes)

    SEG0 = 40 * CHUNK if EW >= 40 * CHUNK else (EW // (2 * CHUNK)) * CHUNK
    rest = ((EW - SEG0) // CHUNK) * CHUNK
    segs = [n for n in (SEG0, rest) if n > 0]
    tail = EW - sum(segs)

    @functools.partial(
        pl.kernel,
        out_type=jax.ShapeDtypeStruct((NC * n_nodes, D), jnp.float32),
        mesh=_sc_mesh(),
        scratch_types=[
            pltpu.VMEM((max(segs),), jnp.int32),
            pltpu.VMEM((max(segs),), jnp.int32),
            pltpu.VMEM((max(tail, 8),), jnp.int32),
            pltpu.VMEM((max(tail, 8),), jnp.int32),
            pltpu.VMEM((CHUNK, D), jnp.float32),
            pltpu.VMEM((CHUNK, D), jnp.float32),
            pltpu.VMEM((max(tail, 8), D), jnp.float32),
            pltpu.VMEM_SHARED((n_nodes, D), jnp.float32),
            pltpu.SemaphoreType.DMA,
            pltpu.SemaphoreType.DMA,
            pltpu.SemaphoreType.DMA,
            pltpu.SemaphoreType.DMA,
        ],
    )
    def k(src_h, dst_h, y_h, zeros_h, out_h,
          sall, dall, sidx_t, didx_t, rows0, rows1, rows_t,
          acc, sg0, sg1, ss0, ss1):
        c = lax.axis_index("c")
        s = lax.axis_index("s")
        base = (c * NS + s) * EW
        r0 = s * per

        def init_from(src0):
            @pl.when(s < NS - 1)
            def _():
                pltpu.sync_copy(src0.at[pl.ds(r0, per)], acc.at[pl.ds(r0, per)])

            @pl.when(s == NS - 1)
            def _():
                pltpu.sync_copy(
                    src0.at[pl.ds((NS - 1) * per, last)],
                    acc.at[pl.ds((NS - 1) * per, last)],
                )

        @pl.when(c == 0)
        def _():
            init_from(y_h)

        @pl.when(c == 1)
        def _():
            init_from(zeros_h)

        plsc.subcore_barrier()

        def sidx(i):
            return sall.at[pl.ds(pl.multiple_of(i * CHUNK, 8), CHUNK)]

        def didx(i):
            return dall.at[pl.ds(pl.multiple_of(i * CHUNK, 8), CHUNK)]

        seg_off = 0
        for seg_n in segs:
            boff = pl.multiple_of(base + seg_off, 8)
            pltpu.sync_copy(src_h.at[pl.ds(boff, seg_n)], sall.at[pl.ds(0, seg_n)])
            pltpu.sync_copy(dst_h.at[pl.ds(boff, seg_n)], dall.at[pl.ds(0, seg_n)])
            n_pair = seg_n // (2 * CHUNK)

            pltpu.async_copy(y_h.at[sidx(0)], rows0, sg0)

            def body(j, carry):
                a = 2 * j

                @pl.when(j > 0)
                def _():
                    pltpu.make_async_copy(rows1, acc.at[didx(a - 1)], ss1).wait()

                pltpu.make_async_copy(y_h.at[sidx(a)], rows0, sg0).wait()
                pltpu.async_copy(y_h.at[sidx(a + 1)], rows1, sg1)
                pltpu.async_copy(rows0, acc.at[didx(a)], ss0, add=True)
                pltpu.make_async_copy(y_h.at[sidx(a + 1)], rows1, sg1).wait()
                pltpu.async_copy(rows1, acc.at[didx(a + 1)], ss1, add=True)
                pltpu.make_async_copy(rows0, acc.at[didx(a)], ss0).wait()

                @pl.when(j < n_pair - 1)
                def _():
                    pltpu.async_copy(y_h.at[sidx(a + 2)], rows0, sg0)

                return carry

            lax.fori_loop(0, n_pair, body, 0)
            pltpu.make_async_copy(
                rows1, acc.at[didx(seg_n // CHUNK - 1)], ss1
            ).wait()
            seg_off += seg_n

        if tail:
            off = pl.multiple_of(base + seg_off, 8)
            pltpu.sync_copy(src_h.at[pl.ds(off, tail)], sidx_t.at[pl.ds(0, tail)])
            pltpu.sync_copy(dst_h.at[pl.ds(off, tail)], didx_t.at[pl.ds(0, tail)])
            pltpu.async_copy(
                y_h.at[sidx_t.at[pl.ds(0, tail)]], rows_t.at[pl.ds(0, tail)], sg0
            ).wait()
            pltpu.sync_copy(
                rows_t.at[pl.ds(0, tail)], acc.at[didx_t.at[pl.ds(0, tail)]], add=True
            )

        plsc.subcore_barrier()

        @pl.when(s < NS - 1)
        def _():
            pltpu.sync_copy(
                acc.at[pl.ds(r0, per)], out_h.at[pl.ds(c * n_nodes + r0, per)]
            )

        @pl.when(s == NS - 1)
        def _():
            pltpu.sync_copy(
                acc.at[pl.ds((NS - 1) * per, last)],
                out_h.at[pl.ds(c * n_nodes + (NS - 1) * per, last)],
            )

    return k(src, dst, y, zeros)


def _tc_prescale(deg16, x):
    n, d = x.shape
    R = 2000

    def body(deg_ref, x_ref, y_ref):
        dinv = lax.rsqrt(deg_ref[...] + 1.0)
        y_ref[...] = x_ref[...] * dinv[:, 0:1]

    return pl.pallas_call(
        body,
        grid=(n // R,),
        in_specs=[
            pl.BlockSpec((R, DEGW), lambda i: (i, 0)),
            pl.BlockSpec((R, d), lambda i: (i, 0)),
        ],
        out_specs=pl.BlockSpec((R, d), lambda i: (i, 0)),
        out_shape=jax.ShapeDtypeStruct((n, d), jnp.float32),
    )(deg16, x)


def _tc_dense(s_p, deg16, h, Mz, Bz, Mr, Br, Mh, Bh, Wout, bv):
    n, d = h.shape
    R = 2000

    def body(s_ref, deg_ref, h_ref,
             Mz_ref, Bz_ref, Mr_ref, Br_ref, Mh_ref, Bh_ref, Wout_ref, bv_ref,
             z_ref, h0_ref):
        dinv = lax.rsqrt(deg_ref[...] + 1.0)[:, 0:1]
        hh = h_ref[...]
        agg = (s_ref[0] + s_ref[1]) * dinv
        dot = lambda a, b: jnp.dot(a, b, preferred_element_type=jnp.float32)
        Zg = jax.nn.sigmoid(dot(agg, Mz_ref[...]) + dot(hh, Bz_ref[...]) + bv_ref[0:1])
        Rg = jax.nn.sigmoid(dot(agg, Mr_ref[...]) + dot(hh, Br_ref[...]) + bv_ref[1:2])
        Ht = jnp.tanh(dot(agg, Mh_ref[...]) + dot(hh * Rg, Bh_ref[...]) + bv_ref[2:3])
        h0 = Zg * hh + (1.0 - Zg) * Ht
        h0_ref[...] = h0
        z_ref[...] = dot(jnp.maximum(h0, 0.0), Wout_ref[...]) + bv_ref[3:4]

    wspec = pl.BlockSpec((d, d), lambda i: (0, 0))
    return pl.pallas_call(
        body,
        grid=(n // R,),
        in_specs=[
            pl.BlockSpec((2, R, d), lambda i: (0, i, 0)),
            pl.BlockSpec((R, DEGW), lambda i: (i, 0)),
            pl.BlockSpec((R, d), lambda i: (i, 0)),
            wspec, wspec, wspec, wspec, wspec, wspec, wspec,
            pl.BlockSpec((4, d), lambda i: (0, 0)),
        ],
        out_specs=[
            pl.BlockSpec((R, d), lambda i: (i, 0)),
            pl.BlockSpec((R, d), lambda i: (i, 0)),
        ],
        out_shape=[
            jax.ShapeDtypeStruct((n, d), jnp.float32),
            jax.ShapeDtypeStruct((n, d), jnp.float32),
        ],
    )(s_p, deg16, h, Mz, Bz, Mr, Br, Mh, Bh, Wout, bv)


def kernel(node_feat, h, src, dst, Wz, bz, Wr, br, Wh, bh,
           Wlz, blz, Wlr, blr, Wlh, blh, Wout, bout):
    n, d = node_feat.shape

    zeros = jnp.zeros((n, d), jnp.float32)

    deg_pk, prp = _sc_degree(dst, n)
    deg = deg_pk.reshape(NC, prp * 128)[:, :n].sum(axis=0)
    deg16 = jnp.broadcast_to(deg[:, None], (n, DEGW))

    y = _tc_prescale(deg16, node_feat)

    s_flat = _sc_aggregate(src, dst, y, zeros, n)
    s_p = s_flat.reshape(NC, n, d)

    Mz = Wz @ Wlz[:d]
    Mr = Wr @ Wlr[:d]
    Mh = Wh @ Wlh[:d]
    bv = jnp.stack([
        bz @ Wlz[:d] + blz,
        br @ Wlr[:d] + blr,
        bh @ Wlh[:d] + blh,
        bout,
    ])

    z, h0 = _tc_dense(s_p, deg16, h, Mz, Wlz[d:], Mr, Wlr[d:], Mh, Wlh[d:],
                      Wout, bv)
    return (z, h0)


# speedup vs baseline: 1.0837x; 1.0312x over previous
import functools

import jax
import jax.numpy as jnp
from jax import lax
from jax.experimental import pallas as pl
from jax.experimental.pallas import tpu as pltpu
from jax.experimental.pallas import tpu_sc as plsc

NC = 2
NS = 16
NW = NC * NS
CHUNK = 128
DEGW = 16


def _sc_mesh():
    return plsc.VectorSubcoreMesh(
        core_axis_name="c", subcore_axis_name="s", num_cores=NC, num_subcores=NS
    )


def _row_split(n_rows):
    per = ((n_rows // NS + 7) // 8) * 8
    last = n_rows - (NS - 1) * per
    return per, last


def _sc_degree(dst, n_nodes):
    E = dst.shape[0]
    EW = E // NW
    UNROLL = 5
    n_vec = EW // 16
    n_grp = n_vec // UNROLL
    rem_vec = n_vec % UNROLL
    PR = -(-n_nodes // 128)
    PRP = ((PR + 15) // 16) * 16
    rows_per_tile = PRP // NS

    @functools.partial(
        pl.kernel,
        out_type=jax.ShapeDtypeStruct((NC * PRP, 128), jnp.float32),
        mesh=_sc_mesh(),
        scratch_types=[
            pltpu.VMEM((EW,), jnp.int32),
            pltpu.VMEM((PRP, 128), jnp.float32),
            pltpu.VMEM((PRP,), jnp.int32),
            pltpu.VMEM_SHARED((PRP, 128), jnp.float32),
            pltpu.SemaphoreType.DMA,
        ],
        compiler_params=pltpu.CompilerParams(needs_layout_passes=False),
    )
    def k(dst_h, out_h, didx, hist, rowidx, acc, sem):
        c = lax.axis_index("c")
        s = lax.axis_index("s")
        base = (c * NS + s) * EW

        pltpu.async_copy(dst_h.at[pl.ds(pl.multiple_of(base, 8), EW)], didx, sem)

        zero16 = jnp.zeros((16,), jnp.float32)

        def zero_body(i, carry):
            hist[i >> 3, pl.ds((i & 7) * 16, 16)] = zero16
            return carry

        lax.fori_loop(0, PRP * 8, zero_body, 0)

        def iota_body(i, carry):
            rowidx[pl.ds(i * 16, 16)] = lax.iota(jnp.int32, 16) + i * 16
            return carry

        lax.fori_loop(0, PRP // 16, iota_body, 0)

        @pl.when(s == 0)
        def _():
            pltpu.sync_copy(hist, acc)

        plsc.subcore_barrier()
        pltpu.make_async_copy(
            dst_h.at[pl.ds(pl.multiple_of(base, 8), EW)], didx, sem
        ).wait()

        ones16 = jnp.ones((16,), jnp.float32)

        def hist_add(j):
            idx = didx[pl.ds(pl.multiple_of(j * 16, 16), 16)]
            plsc.addupdate_scatter(
                hist,
                [lax.shift_right_logical(idx, 7), lax.bitwise_and(idx, 127)],
                ones16,
            )

        def grp_body(g, carry):
            for u in range(UNROLL):
                hist_add(g * UNROLL + u)
            return carry

        lax.fori_loop(0, n_grp, grp_body, 0)
        for j in range(n_grp * UNROLL, n_vec):
            hist_add(j)

        pltpu.sync_copy(hist, acc.at[rowidx], add=True)
        plsc.subcore_barrier()

        ndump = PRP // 8

        @pl.when(s < ndump)
        def _():
            r0 = s * 8
            pltpu.sync_copy(
                acc.at[pl.ds(r0, 8)], out_h.at[pl.ds(c * PRP + r0, 8)]
            )

    return k(dst), PRP


def _sc_aggregate(src, dst, y, zeros, n_nodes):
    E = src.shape[0]
    D = y.shape[1]
    EW = E // NW
    n_full = EW // CHUNK
    tail = EW % CHUNK
    per, last = _row_split(n_nodes)

    SEG0 = 40 * CHUNK if EW >= 40 * CHUNK else (EW // (2 * CHUNK)) * CHUNK
    rest = ((EW - SEG0) // CHUNK) * CHUNK
    segs = [n for n in (SEG0, rest) if n > 0]
    tail = EW - sum(segs)

    @functools.partial(
        pl.kernel,
        out_type=jax.ShapeDtypeStruct((NC * n_nodes, D), jnp.float32),
        mesh=_sc_mesh(),
        scratch_types=[
            pltpu.VMEM((max(segs),), jnp.int32),
            pltpu.VMEM((max(segs),), jnp.int32),
            pltpu.VMEM((max(tail, 8),), jnp.int32),
            pltpu.VMEM((max(tail, 8),), jnp.int32),
            pltpu.VMEM((CHUNK, D), jnp.float32),
            pltpu.VMEM((CHUNK, D), jnp.float32),
            pltpu.VMEM((max(tail, 8), D), jnp.float32),
            pltpu.VMEM_SHARED((n_nodes, D), jnp.float32),
            pltpu.SemaphoreType.DMA,
            pltpu.SemaphoreType.DMA,
            pltpu.SemaphoreType.DMA,
            pltpu.SemaphoreType.DMA,
        ],
    )
    def k(src_h, dst_h, y_h, zeros_h, out_h,
          sall, dall, sidx_t, didx_t, rows0, rows1, rows_t,
          acc, sg0, sg1, ss0, ss1):
        c = lax.axis_index("c")
        s = lax.axis_index("s")
        base = (c * NS + s) * EW
        r0 = s * per

        def init_from(src0):
            @pl.when(s < NS - 1)
            def _():
                pltpu.sync_copy(src0.at[pl.ds(r0, per)], acc.at[pl.ds(r0, per)])

            @pl.when(s == NS - 1)
            def _():
                pltpu.sync_copy(
                    src0.at[pl.ds((NS - 1) * per, last)],
                    acc.at[pl.ds((NS - 1) * per, last)],
                )

        @pl.when(c == 0)
        def _():
            init_from(y_h)

        @pl.when(c == 1)
        def _():
            init_from(zeros_h)

        plsc.subcore_barrier()

        def sidx(i):
            return sall.at[pl.ds(pl.multiple_of(i * CHUNK, 8), CHUNK)]

        def didx(i):
            return dall.at[pl.ds(pl.multiple_of(i * CHUNK, 8), CHUNK)]

        seg_off = 0
        for seg_n in segs:
            boff = pl.multiple_of(base + seg_off, 8)
            pltpu.sync_copy(src_h.at[pl.ds(boff, seg_n)], sall.at[pl.ds(0, seg_n)])
            pltpu.sync_copy(dst_h.at[pl.ds(boff, seg_n)], dall.at[pl.ds(0, seg_n)])
            n_pair = seg_n // (2 * CHUNK)

            pltpu.async_copy(y_h.at[sidx(0)], rows0, sg0)

            def body(j, carry):
                a = 2 * j

                @pl.when(j > 0)
                def _():
                    pltpu.make_async_copy(rows1, acc.at[didx(a - 1)], ss1).wait()

                pltpu.make_async_copy(y_h.at[sidx(a)], rows0, sg0).wait()
                pltpu.async_copy(y_h.at[sidx(a + 1)], rows1, sg1)
                pltpu.async_copy(rows0, acc.at[didx(a)], ss0, add=True)
                pltpu.make_async_copy(y_h.at[sidx(a + 1)], rows1, sg1).wait()
                pltpu.async_copy(rows1, acc.at[didx(a + 1)], ss1, add=True)
                pltpu.make_async_copy(rows0, acc.at[didx(a)], ss0).wait()

                @pl.when(j < n_pair - 1)
                def _():
                    pltpu.async_copy(y_h.at[sidx(a + 2)], rows0, sg0)

                return carry

            lax.fori_loop(0, n_pair, body, 0)
            pltpu.make_async_copy(
                rows1, acc.at[didx(seg_n // CHUNK - 1)], ss1
            ).wait()
            seg_off += seg_n

        if tail:
            off = pl.multiple_of(base + seg_off, 8)
            pltpu.sync_copy(src_h.at[pl.ds(off, tail)], sidx_t.at[pl.ds(0, tail)])
            pltpu.sync_copy(dst_h.at[pl.ds(off, tail)], didx_t.at[pl.ds(0, tail)])
            pltpu.async_copy(
                y_h.at[sidx_t.at[pl.ds(0, tail)]], rows_t.at[pl.ds(0, tail)], sg0
            ).wait()
            pltpu.sync_copy(
                rows_t.at[pl.ds(0, tail)], acc.at[didx_t.at[pl.ds(0, tail)]], add=True
            )

        plsc.subcore_barrier()

        @pl.when(s < NS - 1)
        def _():
            pltpu.sync_copy(
                acc.at[pl.ds(r0, per)], out_h.at[pl.ds(c * n_nodes + r0, per)]
            )

        @pl.when(s == NS - 1)
        def _():
            pltpu.sync_copy(
                acc.at[pl.ds((NS - 1) * per, last)],
                out_h.at[pl.ds(c * n_nodes + (NS - 1) * per, last)],
            )

    return k(src, dst, y, zeros)


def _tc_prescale(deg16, x):
    n, d = x.shape
    R = 2000

    def body(deg_ref, x_ref, y_ref):
        dinv = lax.rsqrt(deg_ref[...] + 1.0)
        y_ref[...] = x_ref[...] * dinv[:, 0:1]

    return pl.pallas_call(
        body,
        grid=(n // R,),
        in_specs=[
            pl.BlockSpec((R, DEGW), lambda i: (i, 0)),
            pl.BlockSpec((R, d), lambda i: (i, 0)),
        ],
        out_specs=pl.BlockSpec((R, d), lambda i: (i, 0)),
        out_shape=jax.ShapeDtypeStruct((n, d), jnp.float32),
    )(deg16, x)


def _tc_dense(s_p, deg16, h, Mz, Bz, Mr, Br, Mh, Bh, Wout, bv):
    n, d = h.shape
    R = 2000

    def body(s_ref, deg_ref, h_ref,
             Mz_ref, Bz_ref, Mr_ref, Br_ref, Mh_ref, Bh_ref, Wout_ref, bv_ref,
             z_ref, h0_ref):
        dinv = lax.rsqrt(deg_ref[...] + 1.0)[:, 0:1]
        hh = h_ref[...]
        agg = (s_ref[0] + s_ref[1]) * dinv
        dot = lambda a, b: jnp.dot(a, b, preferred_element_type=jnp.float32)
        Zg = jax.nn.sigmoid(dot(agg, Mz_ref[...]) + dot(hh, Bz_ref[...]) + bv_ref[0:1])
        Rg = jax.nn.sigmoid(dot(agg, Mr_ref[...]) + dot(hh, Br_ref[...]) + bv_ref[1:2])
        Ht = jnp.tanh(dot(agg, Mh_ref[...]) + dot(hh * Rg, Bh_ref[...]) + bv_ref[2:3])
        h0 = Zg * hh + (1.0 - Zg) * Ht
        h0_ref[...] = h0
        z_ref[...] = dot(jnp.maximum(h0, 0.0), Wout_ref[...]) + bv_ref[3:4]

    wspec = pl.BlockSpec((d, d), lambda i: (0, 0))
    return pl.pallas_call(
        body,
        grid=(n // R,),
        in_specs=[
            pl.BlockSpec((2, R, d), lambda i: (0, i, 0)),
            pl.BlockSpec((R, DEGW), lambda i: (i, 0)),
            pl.BlockSpec((R, d), lambda i: (i, 0)),
            wspec, wspec, wspec, wspec, wspec, wspec, wspec,
            pl.BlockSpec((4, d), lambda i: (0, 0)),
        ],
        out_specs=[
            pl.BlockSpec((R, d), lambda i: (i, 0)),
            pl.BlockSpec((R, d), lambda i: (i, 0)),
        ],
        out_shape=[
            jax.ShapeDtypeStruct((n, d), jnp.float32),
            jax.ShapeDtypeStruct((n, d), jnp.float32),
        ],
    )(s_p, deg16, h, Mz, Bz, Mr, Br, Mh, Bh, Wout, bv)


def kernel(node_feat, h, src, dst, Wz, bz, Wr, br, Wh, bh,
           Wlz, blz, Wlr, blr, Wlh, blh, Wout, bout):
    n, d = node_feat.shape

    zeros = jnp.zeros((n, d), jnp.float32)

    deg_pk, prp = _sc_degree(dst, n)
    deg = deg_pk.reshape(NC, prp * 128)[:, :n].sum(axis=0)
    deg16 = jnp.broadcast_to(deg[:, None], (n, DEGW))

    y = _tc_prescale(deg16, node_feat)

    s_flat = _sc_aggregate(src, dst, y, zeros, n)
    s_p = s_flat.reshape(NC, n, d)

    Mz = Wz @ Wlz[:d]
    Mr = Wr @ Wlr[:d]
    Mh = Wh @ Wlh[:d]
    bv = jnp.stack([
        bz @ Wlz[:d] + blz,
        br @ Wlr[:d] + blr,
        bh @ Wlh[:d] + blh,
        bout,
    ])

    z, h0 = _tc_dense(s_p, deg16, h, Mz, Wlz[d:], Mr, Wlr[d:], Mh, Wlh[d:],
                      Wout, bv)
    return (z, h0)
